# Initial kernel scaffold; baseline (speedup 1.0000x reference)
import jax
import jax.numpy as jnp
from jax import lax
from jax.experimental import pallas as pl
from jax.experimental.pallas import tpu as pltpu
from jax.experimental.pallas import tpu_sc as plsc

N_FIELDS_K = 26
N_CAT = 100000
B_TOTAL = 16384
NW = 32
B_PER = B_TOTAL // NW
PER_W = B_PER * N_FIELDS_K
CHUNK = 128
N_CHUNKS = PER_W // CHUNK
PERIOD = 208
N_PAT = PERIOD // 16
K_PER_PAT = PER_W // PERIOD


def _body(x_hbm, w_hbm, bias_hbm, pat_hbm, out_hbm,
          idx_v, vals_v, acc_v, pat_v, bias_v, sem):
    c = lax.axis_index("c")
    s = lax.axis_index("s")
    wid = s * 2 + c

    pltpu.sync_copy(x_hbm.at[wid], idx_v)
    pltpu.sync_copy(pat_hbm, pat_v)
    pltpu.sync_copy(bias_hbm, bias_v)

    for c13 in range(N_PAT):
        voff = pat_v[c13]

        def add_off(k, _, c13=c13, voff=voff):
            st = k * PERIOD + c13 * 16
            idx_v[pl.ds(st, 16)] = idx_v[pl.ds(st, 16)] + voff
            return 0

        lax.fori_loop(0, K_PER_PAT, add_off, 0)

    def fire(j, _):
        pltpu.async_copy(
            w_hbm.at[idx_v.at[pl.ds(j * CHUNK, CHUNK)]],
            vals_v.at[pl.ds(j * CHUNK, CHUNK)], sem)
        return 0

    lax.fori_loop(0, N_CHUNKS, fire, 0)

    def drain(j, _):
        pltpu.make_async_copy(
            w_hbm.at[idx_v.at[pl.ds(0, CHUNK)]],
            vals_v.at[pl.ds(j * CHUNK, CHUNK)], sem).wait()
        return 0

    lax.fori_loop(0, N_CHUNKS, drain, 0)

    stride_iota = lax.iota(jnp.int32, 16) * N_FIELDS_K
    b0 = bias_v[0]

    def reduce_chunk(i, _):
        basep = i * (16 * N_FIELDS_K)
        acc = jnp.full((16,), b0, dtype=jnp.float32)
        for f in range(N_FIELDS_K):
            acc = acc + plsc.load_gather(vals_v, [stride_iota + (basep + f)])
        acc_v[pl.ds(i * 16, 16)] = acc
        return 0

    lax.fori_loop(0, B_PER // 16, reduce_chunk, 0)

    pltpu.sync_copy(acc_v, out_hbm.at[wid])


@jax.jit
def _cat_linear(x3, w_flat, bias, pat):
    mesh = plsc.VectorSubcoreMesh(core_axis_name="c", subcore_axis_name="s")
    f = pl.kernel(
        _body,
        out_type=jax.ShapeDtypeStruct((NW, B_PER), jnp.float32),
        mesh=mesh,
        scratch_types=[
            pltpu.VMEM((PER_W,), jnp.int32),
            pltpu.VMEM((PER_W,), jnp.float32),
            pltpu.VMEM((B_PER,), jnp.float32),
            pltpu.VMEM((N_PAT, 16), jnp.int32),
            pltpu.VMEM((1,), jnp.float32),
            pltpu.SemaphoreType.DMA,
        ],
    )
    return f(x3, w_flat, bias, pat)


def kernel(x_cat, W, bias):
    x3 = x_cat.reshape(NW, PER_W)
    w_flat = W.reshape(-1)
    pos = jnp.arange(PERIOD, dtype=jnp.int32)
    pat = ((pos % N_FIELDS_K) * N_CAT).reshape(N_PAT, 16)
    out = _cat_linear(x3, w_flat, bias, pat)
    return out.reshape(B_TOTAL, 1)


# speedup vs baseline: 1.1278x; 1.1278x over previous
import jax
import jax.numpy as jnp
from jax import lax
from jax.experimental import pallas as pl
from jax.experimental.pallas import tpu as pltpu
from jax.experimental.pallas import tpu_sc as plsc

N_FIELDS_K = 26
N_CAT = 100000
B_TOTAL = 16384
NW = 32
B_PER = B_TOTAL // NW
PER_W = B_PER * N_FIELDS_K
CHUNK = 128
N_CHUNKS = PER_W // CHUNK
PERIOD = 208
N_PAT = PERIOD // 16
K_PER_PAT = PER_W // PERIOD


def _body(x_hbm, w_hbm, bias_hbm, pat_hbm, out_hbm,
          idx_v, vals_v, acc_v, pat_v, bias_v, sem):
    c = lax.axis_index("c")
    s = lax.axis_index("s")
    wid = s * 2 + c

    pltpu.sync_copy(x_hbm.at[wid], idx_v)
    pltpu.sync_copy(pat_hbm, pat_v)
    pltpu.sync_copy(bias_hbm, bias_v)

    for c13 in range(N_PAT):
        voff = pat_v[c13]

        def add_off(k, _, c13=c13, voff=voff):
            st = k * PERIOD + c13 * 16
            idx_v[pl.ds(st, 16)] = idx_v[pl.ds(st, 16)] + voff
            return 0

        lax.fori_loop(0, K_PER_PAT, add_off, 0)

    def fire(j, _):
        pltpu.async_copy(
            w_hbm.at[idx_v.at[pl.ds(j * CHUNK, CHUNK)]],
            vals_v.at[pl.ds(j * CHUNK, CHUNK)], sem)
        return 0

    lax.fori_loop(0, N_CHUNKS, fire, 0)

    def drain(j, _):
        pltpu.make_async_copy(
            w_hbm.at[idx_v.at[pl.ds(0, CHUNK)]],
            vals_v.at[pl.ds(j * CHUNK, CHUNK)], sem).wait()
        return 0

    lax.fori_loop(0, N_CHUNKS, drain, 0)

    stride_iota = lax.iota(jnp.int32, 16) * N_FIELDS_K
    vbias = bias_v[pl.ds(0, 16)]

    def reduce_chunk(i, _):
        basep = i * (16 * N_FIELDS_K)
        acc = vbias
        for f in range(N_FIELDS_K):
            acc = acc + plsc.load_gather(vals_v, [stride_iota + (basep + f)])
        acc_v[pl.ds(i * 16, 16)] = acc
        return 0

    lax.fori_loop(0, B_PER // 16, reduce_chunk, 0)

    pltpu.sync_copy(acc_v, out_hbm.at[wid])


@jax.jit
def _cat_linear(x3, w_flat, bias, pat):
    mesh = plsc.VectorSubcoreMesh(core_axis_name="c", subcore_axis_name="s")
    f = pl.kernel(
        _body,
        out_type=jax.ShapeDtypeStruct((NW, B_PER), jnp.float32),
        mesh=mesh,
        compiler_params=pltpu.CompilerParams(needs_layout_passes=False),
        scratch_types=[
            pltpu.VMEM((PER_W,), jnp.int32),
            pltpu.VMEM((PER_W,), jnp.float32),
            pltpu.VMEM((B_PER,), jnp.float32),
            pltpu.VMEM((N_PAT, 16), jnp.int32),
            pltpu.VMEM((16,), jnp.float32),
            pltpu.SemaphoreType.DMA,
        ],
    )
    return f(x3, w_flat, bias, pat)


def kernel(x_cat, W, bias):
    x3 = x_cat.reshape(NW, PER_W)
    w_flat = W.reshape(-1)
    pos = jnp.arange(PERIOD, dtype=jnp.int32)
    pat = ((pos % N_FIELDS_K) * N_CAT).reshape(N_PAT, 16)
    bias16 = jnp.broadcast_to(bias.reshape(()), (16,)).astype(jnp.float32)
    out = _cat_linear(x3, w_flat, bias16, pat)
    return out.reshape(B_TOTAL, 1)


# speedup vs baseline: 1.1518x; 1.0213x over previous
import jax
import jax.numpy as jnp
from jax import lax
from jax.experimental import pallas as pl
from jax.experimental.pallas import tpu as pltpu
from jax.experimental.pallas import tpu_sc as plsc

N_FIELDS_K = 26
N_CAT = 100000
B_TOTAL = 16384
NW = 32
B_PER = B_TOTAL // NW
PER_W = B_PER * N_FIELDS_K
CHUNK = 104
PERIOD = 208
N_PAT = PERIOD // 16
K_PER_PAT = PER_W // PERIOD


def _body(x_hbm, w_hbm, bias_hbm, pat_hbm, out_hbm,
          idx_v, vals_v, acc_v, pat_v, bias_v, sem):
    c = lax.axis_index("c")
    s = lax.axis_index("s")
    wid = s * 2 + c

    pltpu.sync_copy(x_hbm.at[wid], idx_v)
    pltpu.sync_copy(pat_hbm, pat_v)
    pltpu.sync_copy(bias_hbm, bias_v)

    voffs = [pat_v[c13] for c13 in range(N_PAT)]

    def add_and_fire(k, _):
        base = k * PERIOD
        for c13 in range(N_PAT):
            st = base + c13 * 16
            idx_v[pl.ds(st, 16)] = idx_v[pl.ds(st, 16)] + voffs[c13]
        for half in range(2):
            st = base + half * CHUNK
            pltpu.async_copy(
                w_hbm.at[idx_v.at[pl.ds(st, CHUNK)]],
                vals_v.at[pl.ds(st, CHUNK)], sem)
        return 0

    lax.fori_loop(0, K_PER_PAT, add_and_fire, 0)

    def drain(j, _):
        pltpu.make_async_copy(
            w_hbm.at[idx_v.at[pl.ds(0, CHUNK)]],
            vals_v.at[pl.ds(j * CHUNK, CHUNK)], sem).wait()
        return 0

    lax.fori_loop(0, 2 * K_PER_PAT, drain, 0)

    stride_iota = lax.iota(jnp.int32, 16) * N_FIELDS_K
    vbias = bias_v[pl.ds(0, 16)]

    def reduce_chunk(i, _):
        basep = i * (16 * N_FIELDS_K)
        acc = vbias
        for f in range(N_FIELDS_K):
            acc = acc + plsc.load_gather(vals_v, [stride_iota + (basep + f)])
        acc_v[pl.ds(i * 16, 16)] = acc
        return 0

    lax.fori_loop(0, B_PER // 16, reduce_chunk, 0)

    pltpu.sync_copy(acc_v, out_hbm.at[wid])


@jax.jit
def _cat_linear(x3, w_flat, bias, pat):
    mesh = plsc.VectorSubcoreMesh(core_axis_name="c", subcore_axis_name="s")
    f = pl.kernel(
        _body,
        out_type=jax.ShapeDtypeStruct((NW, B_PER), jnp.float32),
        mesh=mesh,
        compiler_params=pltpu.CompilerParams(
            needs_layout_passes=False,
            skip_device_barrier=True,
            disable_bounds_checks=True,
            disable_semaphore_checks=True,
        ),
        scratch_types=[
            pltpu.VMEM((PER_W,), jnp.int32),
            pltpu.VMEM((PER_W,), jnp.float32),
            pltpu.VMEM((B_PER,), jnp.float32),
            pltpu.VMEM((N_PAT, 16), jnp.int32),
            pltpu.VMEM((16,), jnp.float32),
            pltpu.SemaphoreType.DMA,
        ],
    )
    return f(x3, w_flat, bias, pat)


def kernel(x_cat, W, bias):
    x3 = x_cat.reshape(NW, PER_W)
    w_flat = W.reshape(-1)
    pos = jnp.arange(PERIOD, dtype=jnp.int32)
    pat = ((pos % N_FIELDS_K) * N_CAT).reshape(N_PAT, 16)
    bias16 = jnp.broadcast_to(bias.reshape(()), (16,)).astype(jnp.float32)
    out = _cat_linear(x3, w_flat, bias16, pat)
    return out.reshape(B_TOTAL, 1)


# speedup vs baseline: 1.2751x; 1.1071x over previous
import jax
import jax.numpy as jnp
from jax import lax
from jax.experimental import pallas as pl
from jax.experimental.pallas import tpu as pltpu
from jax.experimental.pallas import tpu_sc as plsc

N_FIELDS_K = 26
N_CAT = 100000
B_TOTAL = 16384
NW = 32
B_PER = B_TOTAL // NW
CHUNK = 128
N_BLK = B_PER // CHUNK


def _body(x_hbm, w_hbm, bias_hbm, out_hbm, idx_v, vals_v, acc_v, bias_v, sem):
    c = lax.axis_index("c")
    s = lax.axis_index("s")
    wid = s * 2 + c
    base = wid * B_PER

    pltpu.sync_copy(x_hbm.at[:, pl.ds(base, B_PER)], idx_v)
    pltpu.sync_copy(bias_hbm, bias_v)

    def fire(j, _):
        f = j // N_BLK
        blk = (j % N_BLK) * CHUNK
        pltpu.async_copy(
            w_hbm.at[0, pl.ds(f * N_CAT, N_CAT)].at[idx_v.at[f, pl.ds(blk, CHUNK)]],
            vals_v.at[f, pl.ds(blk, CHUNK)], sem)
        return 0

    lax.fori_loop(0, N_FIELDS_K * N_BLK, fire, 0)

    def drain(j, _):
        f = j // N_BLK
        blk = (j % N_BLK) * CHUNK
        pltpu.make_async_copy(
            w_hbm.at[0, pl.ds(0, N_CAT)].at[idx_v.at[0, pl.ds(0, CHUNK)]],
            vals_v.at[f, pl.ds(blk, CHUNK)], sem).wait()
        return 0

    lax.fori_loop(0, N_FIELDS_K * N_BLK, drain, 0)

    vbias = bias_v[pl.ds(0, 16)]

    def reduce_chunk(i, _):
        st = i * 16
        acc = vbias
        for f in range(N_FIELDS_K):
            acc = acc + vals_v[f, pl.ds(st, 16)]
        acc_v[pl.ds(st, 16)] = acc
        return 0

    lax.fori_loop(0, B_PER // 16, reduce_chunk, 0)

    pltpu.sync_copy(acc_v, out_hbm.at[pl.ds(base, B_PER)])


@jax.jit
def _cat_linear(x_t, w_row, bias16):
    mesh = plsc.VectorSubcoreMesh(core_axis_name="c", subcore_axis_name="s")
    f = pl.kernel(
        _body,
        out_type=jax.ShapeDtypeStruct((B_TOTAL,), jnp.float32),
        mesh=mesh,
        compiler_params=pltpu.CompilerParams(
            needs_layout_passes=False,
            skip_device_barrier=True,
            disable_bounds_checks=True,
            disable_semaphore_checks=True,
            use_tc_tiling_on_sc=False,
        ),
        scratch_types=[
            pltpu.VMEM((N_FIELDS_K, B_PER), jnp.int32),
            pltpu.VMEM((N_FIELDS_K, B_PER), jnp.float32),
            pltpu.VMEM((B_PER,), jnp.float32),
            pltpu.VMEM((16,), jnp.float32),
            pltpu.SemaphoreType.DMA,
        ],
    )
    return f(x_t, w_row, bias16)


def kernel(x_cat, W, bias):
    x_t = x_cat.T
    w_row = W.T
    bias16 = jnp.broadcast_to(bias.reshape(()), (16,)).astype(jnp.float32)
    out = _cat_linear(x_t, w_row, bias16)
    return out.reshape(B_TOTAL, 1)


# speedup vs baseline: 3.8669x; 3.0326x over previous
import jax
import jax.numpy as jnp
from jax import lax
from jax.experimental import pallas as pl
from jax.experimental.pallas import tpu as pltpu
from jax.experimental.pallas import tpu_sc as plsc

N_FIELDS_K = 26
N_CAT = 100000
B_TOTAL = 16384
NW = 32
B_PER = B_TOTAL // NW
CHUNK = 128
N_BLK = B_PER // CHUNK
TAIL = 64
SPLIT = N_FIELDS_K * N_CAT - TAIL
LAST_LEN = N_CAT - TAIL


def _body(x_hbm, wm_hbm, wt_hbm, bias_hbm, out_hbm,
          idx_v, vals_v, acc_v, idx25_v, tail_v, bias_v, sem):
    c = lax.axis_index("c")
    s = lax.axis_index("s")
    wid = s * 2 + c
    base = wid * B_PER

    pltpu.sync_copy(x_hbm.at[:, pl.ds(base, B_PER)], idx_v)
    pltpu.sync_copy(bias_hbm, bias_v)
    pltpu.sync_copy(wt_hbm, tail_v)

    def save_clamp(i, _):
        st = i * 16
        v = idx_v[25, pl.ds(st, 16)]
        idx25_v[pl.ds(st, 16)] = v
        idx_v[25, pl.ds(st, 16)] = jnp.minimum(v, LAST_LEN - 1)
        return 0

    lax.fori_loop(0, B_PER // 16, save_clamp, 0)

    def fire(j, _):
        f = j // N_BLK
        blk = (j % N_BLK) * CHUNK
        pltpu.async_copy(
            wm_hbm.at[0, pl.ds(f * N_CAT, N_CAT)].at[idx_v.at[f, pl.ds(blk, CHUNK)]],
            vals_v.at[f, pl.ds(blk, CHUNK)], sem)
        return 0

    lax.fori_loop(0, (N_FIELDS_K - 1) * N_BLK, fire, 0)

    for b in range(N_BLK):
        pltpu.async_copy(
            wm_hbm.at[0, pl.ds(25 * N_CAT, LAST_LEN)].at[idx_v.at[25, pl.ds(b * CHUNK, CHUNK)]],
            vals_v.at[25, pl.ds(b * CHUNK, CHUNK)], sem)

    def drain(j, _):
        f = j // N_BLK
        blk = (j % N_BLK) * CHUNK
        pltpu.make_async_copy(
            wm_hbm.at[0, pl.ds(0, N_CAT)].at[idx_v.at[0, pl.ds(0, CHUNK)]],
            vals_v.at[f, pl.ds(blk, CHUNK)], sem).wait()
        return 0

    lax.fori_loop(0, N_FIELDS_K * N_BLK, drain, 0)

    vbias = bias_v[pl.ds(0, 16)]

    def reduce_chunk(i, _):
        st = i * 16
        acc = vbias
        for f in range(N_FIELDS_K - 1):
            acc = acc + vals_v[f, pl.ds(st, 16)]
        iv = idx25_v[pl.ds(st, 16)]
        in_tail = iv >= LAST_LEN
        tfix = plsc.load_gather(
            tail_v, [jnp.maximum(iv - LAST_LEN, 0)])
        acc = acc + jnp.where(in_tail, tfix, vals_v[25, pl.ds(st, 16)])
        acc_v[pl.ds(st, 16)] = acc
        return 0

    lax.fori_loop(0, B_PER // 16, reduce_chunk, 0)

    pltpu.sync_copy(acc_v, out_hbm.at[pl.ds(base, B_PER)])


@jax.jit
def _cat_linear(x_t, w_main, w_tail, bias16):
    mesh = plsc.VectorSubcoreMesh(core_axis_name="c", subcore_axis_name="s")
    f = pl.kernel(
        _body,
        out_type=jax.ShapeDtypeStruct((B_TOTAL,), jnp.float32),
        mesh=mesh,
        compiler_params=pltpu.CompilerParams(
            needs_layout_passes=False,
            skip_device_barrier=True,
            disable_bounds_checks=True,
            disable_semaphore_checks=True,
            use_tc_tiling_on_sc=False,
        ),
        scratch_types=[
            pltpu.VMEM((N_FIELDS_K, B_PER), jnp.int32),
            pltpu.VMEM((N_FIELDS_K, B_PER), jnp.float32),
            pltpu.VMEM((B_PER,), jnp.float32),
            pltpu.VMEM((B_PER,), jnp.int32),
            pltpu.VMEM((TAIL,), jnp.float32),
            pltpu.VMEM((16,), jnp.float32),
            pltpu.SemaphoreType.DMA,
        ],
    )
    return f(x_t, w_main, w_tail, bias16)


def kernel(x_cat, W, bias):
    x_t = x_cat.T
    w_row = W.T
    w_main = w_row[:, :SPLIT]
    w_tail = W[SPLIT:, 0]
    bias16 = jnp.broadcast_to(bias.reshape(()), (16,)).astype(jnp.float32)
    out = _cat_linear(x_t, w_main, w_tail, bias16)
    return out.reshape(B_TOTAL, 1)
